# baseline (speedup 1.0000x reference)
import functools

import jax
import jax.numpy as jnp
from jax import lax
from jax.experimental import pallas as pl
from jax.experimental.pallas import tpu as pltpu
from jax.experimental.pallas import tpu_sc as plsc

NBUF = 4
RING = 8
LEAD = 4
LANES = 16
TILE_L = 128


@functools.cache
def _build_chunks(B, D, VU):
    info = plsc.get_sparse_core_info()
    NC, NS = info.num_cores, info.num_subcores
    NW = NC * NS
    bpw = B // NW

    mesh = plsc.VectorSubcoreMesh(core_axis_name="c", subcore_axis_name="s")

    @functools.partial(
        pl.kernel,
        mesh=mesh,
        compiler_params=pltpu.CompilerParams(use_tc_tiling_on_sc=True),
        out_type=jax.ShapeDtypeStruct((B, D, TILE_L), jnp.float32),
        scratch_types=(
            [
                pltpu.VMEM((bpw + LANES,), jnp.int32),
                pltpu.VMEM((RING, D, TILE_L), jnp.float32),
            ]
            + [pltpu.SemaphoreType.DMA for _ in range(2 * RING)]
        ),
    )
    def chunks_k(uid_h, utT_h, c_h, uid_v, chunk, *sems):
        gsem = sems[:RING]
        wsem = sems[RING:]
        wid = lax.axis_index("s") * NC + lax.axis_index("c")
        base_i = wid * bpw
        pltpu.sync_copy(uid_h.at[pl.ds(base_i, bpw)],
                        uid_v.at[pl.ds(0, bpw)])

        def col(i):
            return (uid_v[pl.ds(i, LANES)][0] // TILE_L) * TILE_L

        for j in range(LEAD):
            pltpu.async_copy(utT_h.at[:, pl.ds(col(j), TILE_L)],
                             chunk.at[j], gsem[j])

        def step(i, carry):
            j_dyn = lax.rem(i, RING)
            for j in range(RING):

                @pl.when(j_dyn == j)
                def _():
                    pltpu.make_async_copy(
                        utT_h.at[:, pl.ds(0, TILE_L)], chunk.at[j],
                        gsem[j]).wait()
                    pltpu.async_copy(chunk.at[j], c_h.at[base_i + i],
                                     wsem[j])
                    jn = (j + LEAD) % RING

                    @pl.when(i >= LEAD)
                    def _():
                        pltpu.make_async_copy(
                            chunk.at[jn], c_h.at[base_i], wsem[jn]).wait()

                    @pl.when(i + LEAD < bpw)
                    def _():
                        pltpu.async_copy(
                            utT_h.at[:, pl.ds(col(i + LEAD), TILE_L)],
                            chunk.at[jn], gsem[jn])
            return carry

        lax.fori_loop(0, bpw, step, 0)
        for jj in range(LEAD):
            j = (bpw - LEAD + jj) % RING
            pltpu.make_async_copy(chunk.at[j], c_h.at[base_i],
                                  wsem[j]).wait()

    return chunks_k


@functools.cache
def _build_repack(VC, D):
    W = 1024
    G = -(-((VC + 1) // 2) // W)
    H = G * W

    last_blk = -(-VC // W) - 1

    def body(a_ref, b_ref, o_ref):
        o_ref[:, 0:D] = jnp.swapaxes(a_ref[...], 0, 1)
        o_ref[:, D:2 * D] = jnp.swapaxes(b_ref[...], 0, 1)

    rp = pl.pallas_call(
        body,
        grid=(G,),
        in_specs=[
            pl.BlockSpec((D, W), lambda k: (0, k)),
            pl.BlockSpec((D, W), lambda k: (0, jnp.minimum(k + G, last_blk))),
        ],
        out_specs=pl.BlockSpec((W, 2 * D), lambda k: (k, 0)),
        out_shape=jax.ShapeDtypeStruct((H, 2 * D), jnp.float32),
    )
    return rp, H


@functools.cache
def _build_main(B, T, D, VC, VG, H):
    info = plsc.get_sparse_core_info()
    NC, NS = info.num_cores, info.num_subcores
    NW = NC * NS
    assert B % NW == 0, (B, NW)
    bpw = B // NW
    assert bpw % NBUF == 0
    ngroups = bpw // NBUF
    nk = D // LANES
    nrow = bpw * D // TILE_L
    per_row = TILE_L // D
    nflat = bpw * T

    mesh = plsc.VectorSubcoreMesh(core_axis_name="c", subcore_axis_name="s")

    @functools.partial(
        pl.kernel,
        mesh=mesh,
        compiler_params=pltpu.CompilerParams(use_tc_tiling_on_sc=False),
        out_type=jax.ShapeDtypeStruct((B, T, D), jnp.float32),
        scratch_types=(
            [
                pltpu.VMEM((bpw + LANES,), jnp.int32),
                pltpu.VMEM((bpw,), jnp.int32),
                pltpu.VMEM((bpw, T), jnp.int32),
                pltpu.VMEM((nflat + LANES,), jnp.int32),
                pltpu.VMEM((nrow, TILE_L), jnp.int32),
                pltpu.VMEM((bpw * D,), jnp.float32),
                pltpu.VMEM((bpw, D), jnp.float32),
                pltpu.VMEM((NBUF, T, 2 * D), jnp.float32),
                pltpu.VMEM((NBUF, T, D), jnp.float32),
                pltpu.SemaphoreType.DMA,
                pltpu.SemaphoreType.DMA,
            ]
            + [pltpu.SemaphoreType.DMA for _ in range(2 * NBUF)]
        ),
    )
    def emb(uid_h, gid_h, cid2_h, loff_h, cf_h, ct_h, gt_h, out_h,
            uid_v, gid_v, cid2_v, loff, eidx, s_v, g_v, in_b, out_b,
            sem_g, sem_s, *sems):
        gsem = sems[:NBUF]
        ssem = sems[NBUF:]
        wid = lax.axis_index("s") * NC + lax.axis_index("c")
        base = wid * bpw
        pltpu.sync_copy(uid_h.at[pl.ds(base, bpw)],
                        uid_v.at[pl.ds(0, bpw)])
        pltpu.sync_copy(gid_h.at[pl.ds(base, bpw)], gid_v)
        pltpu.sync_copy(cid2_h.at[pl.ds(base, bpw)], cid2_v)
        pltpu.sync_copy(loff_h.at[pl.ds(base * T, nflat)],
                        loff.at[pl.ds(0, nflat)])
        pltpu.async_copy(gt_h.at[gid_v], g_v, sem_g)

        iot = lax.iota(jnp.int32, LANES)

        def idxrow(r, carry):
            i0 = per_row * r
            for q in range(TILE_L // LANES):
                i_loc = i0 + q // (D // LANES)
                lane = lax.rem(uid_v[pl.ds(i_loc, LANES)][0], TILE_L)
                d_vec = (q % (D // LANES)) * LANES + iot
                eidx[r, pl.ds(q * LANES, LANES)] = (
                    (base + i_loc) * (D * TILE_L) + d_vec * TILE_L + lane
                )
            return carry

        lax.fori_loop(0, nrow, idxrow, 0)

        def efire(r, carry):
            pltpu.async_copy(cf_h.at[eidx.at[r]],
                             s_v.at[pl.ds(r * TILE_L, TILE_L)], sem_s)
            return carry

        def edrain(r, carry):
            pltpu.make_async_copy(cf_h.at[eidx.at[0]],
                                  s_v.at[pl.ds(0, TILE_L)], sem_s).wait()
            return carry

        lax.fori_loop(0, nrow, efire, 0)
        lax.fori_loop(0, nrow, edrain, 0)

        pltpu.make_async_copy(gt_h.at[gid_v], g_v, sem_g).wait()

        def srow(i, carry):
            for k in range(nk):
                s_v[pl.ds(i * D + k * LANES, LANES)] = (
                    s_v[pl.ds(i * D + k * LANES, LANES)]
                    + g_v[i, pl.ds(k * LANES, LANES)]
                )
            return carry

        lax.fori_loop(0, bpw, srow, 0)

        for j in range(NBUF):
            pltpu.async_copy(ct_h.at[cid2_v.at[j]], in_b.at[j], gsem[j])

        def group(gi, carry):
            for j in range(NBUF):
                b = gi * NBUF + j
                pltpu.make_async_copy(
                    ct_h.at[cid2_v.at[b]], in_b.at[j], gsem[j]).wait()

                @pl.when(gi > 0)
                def _():
                    pltpu.make_async_copy(
                        out_b.at[j], out_h.at[base + b], ssem[j]).wait()

                s_regs = [s_v[pl.ds(b * D + k * LANES, LANES)]
                          for k in range(nk)]

                def trow(t, c2):
                    lo = loff[pl.ds(b * T + t, LANES)][0]
                    for k in range(nk):
                        out_b[j, t, pl.ds(k * LANES, LANES)] = (
                            in_b[j, t, pl.ds(lo + k * LANES, LANES)]
                            + s_regs[k]
                        )
                    return c2

                lax.fori_loop(0, T, trow, 0)

                pltpu.async_copy(out_b.at[j], out_h.at[base + b], ssem[j])

                @pl.when(b + NBUF < bpw)
                def _():
                    pltpu.async_copy(
                        ct_h.at[cid2_v.at[b + NBUF]], in_b.at[j], gsem[j])
            return carry

        lax.fori_loop(0, ngroups, group, 0)

        for j in range(NBUF):
            b = (ngroups - 1) * NBUF + j
            pltpu.make_async_copy(
                out_b.at[j], out_h.at[base + b], ssem[j]).wait()

    return emb


def kernel(user_id, creative_id, geo_id, T, user_table, creative_table,
           geo_table):
    del T
    B, t_static = creative_id.shape
    D = user_table.shape[1]
    chunks_k = _build_chunks(B, D, user_table.shape[0])
    uid = user_id.astype(jnp.int32)
    chunks = chunks_k(uid, user_table.T)
    repack, H = _build_repack(creative_table.shape[0], D)
    ct2 = repack(creative_table.T, creative_table.T)
    emb = _build_main(B, t_static, D, creative_table.shape[0],
                      geo_table.shape[0], H)
    cid = creative_id.astype(jnp.int32)
    big = (cid >= H).astype(jnp.int32)
    cid2 = cid - big * H
    loff = (big * D).reshape(-1)
    return emb(
        uid,
        geo_id.astype(jnp.int32),
        cid2,
        loff,
        chunks.reshape(-1),
        ct2,
        geo_table,
    )


# speedup vs baseline: 1.0054x; 1.0054x over previous
import functools

import jax
import jax.numpy as jnp
from jax import lax
from jax.experimental import pallas as pl
from jax.experimental.pallas import tpu as pltpu
from jax.experimental.pallas import tpu_sc as plsc

NBUF = 4
RING = 8
LEAD = 4
LANES = 16
TILE_L = 128


@functools.cache
def _build_chunks(B, D, VU):
    info = plsc.get_sparse_core_info()
    NC, NS = info.num_cores, info.num_subcores
    NW = NC * NS
    bpw = B // NW

    mesh = plsc.VectorSubcoreMesh(core_axis_name="c", subcore_axis_name="s")

    @functools.partial(
        pl.kernel,
        mesh=mesh,
        compiler_params=pltpu.CompilerParams(use_tc_tiling_on_sc=True),
        out_type=jax.ShapeDtypeStruct((B, D, TILE_L), jnp.float32),
        scratch_types=(
            [
                pltpu.VMEM((bpw + LANES,), jnp.int32),
                pltpu.VMEM((RING, D, TILE_L), jnp.float32),
            ]
            + [pltpu.SemaphoreType.DMA for _ in range(2 * RING)]
        ),
    )
    def chunks_k(uid_h, utT_h, c_h, uid_v, chunk, *sems):
        gsem = sems[:RING]
        wsem = sems[RING:]
        wid = lax.axis_index("s") * NC + lax.axis_index("c")
        base_i = wid * bpw
        pltpu.sync_copy(uid_h.at[pl.ds(base_i, bpw)],
                        uid_v.at[pl.ds(0, bpw)])

        def col(i):
            return (uid_v[pl.ds(i, LANES)][0] // TILE_L) * TILE_L

        for j in range(LEAD):
            pltpu.async_copy(utT_h.at[:, pl.ds(col(j), TILE_L)],
                             chunk.at[j], gsem[j])

        def step(i, carry):
            j_dyn = lax.rem(i, RING)
            for j in range(RING):

                @pl.when(j_dyn == j)
                def _():
                    pltpu.make_async_copy(
                        utT_h.at[:, pl.ds(0, TILE_L)], chunk.at[j],
                        gsem[j]).wait()
                    pltpu.async_copy(chunk.at[j], c_h.at[base_i + i],
                                     wsem[j])
                    jn = (j + LEAD) % RING

                    @pl.when(i >= LEAD)
                    def _():
                        pltpu.make_async_copy(
                            chunk.at[jn], c_h.at[base_i], wsem[jn]).wait()

                    @pl.when(i + LEAD < bpw)
                    def _():
                        pltpu.async_copy(
                            utT_h.at[:, pl.ds(col(i + LEAD), TILE_L)],
                            chunk.at[jn], gsem[jn])
            return carry

        lax.fori_loop(0, bpw, step, 0)
        for jj in range(LEAD):
            j = (bpw - LEAD + jj) % RING
            pltpu.make_async_copy(chunk.at[j], c_h.at[base_i],
                                  wsem[j]).wait()

    return chunks_k


@functools.cache
def _build_repack(VC, D):
    W = 1024
    G = -(-((VC + 1) // 2) // W)
    H = G * W

    last_blk = -(-VC // W) - 1

    def body(a_ref, b_ref, o_ref):
        o_ref[:, 0:D] = jnp.swapaxes(a_ref[...], 0, 1)
        o_ref[:, D:2 * D] = jnp.swapaxes(b_ref[...], 0, 1)

    rp = pl.pallas_call(
        body,
        grid=(G,),
        in_specs=[
            pl.BlockSpec((D, W), lambda k: (0, k)),
            pl.BlockSpec((D, W), lambda k: (0, jnp.minimum(k + G, last_blk))),
        ],
        out_specs=pl.BlockSpec((W, 2 * D), lambda k: (k, 0)),
        out_shape=jax.ShapeDtypeStruct((H, 2 * D), jnp.float32),
    )
    return rp, H


@functools.cache
def _build_main(B, T, D, VC, VG, H):
    info = plsc.get_sparse_core_info()
    NC, NS = info.num_cores, info.num_subcores
    NW = NC * NS
    assert B % NW == 0, (B, NW)
    bpw = B // NW
    assert bpw % NBUF == 0
    ngroups = bpw // NBUF
    nk = D // LANES
    nrow = bpw * D // TILE_L
    per_row = TILE_L // D
    nflat = bpw * T

    mesh = plsc.VectorSubcoreMesh(core_axis_name="c", subcore_axis_name="s")

    @functools.partial(
        pl.kernel,
        mesh=mesh,
        compiler_params=pltpu.CompilerParams(use_tc_tiling_on_sc=False),
        out_type=jax.ShapeDtypeStruct((B, T, D), jnp.float32),
        scratch_types=(
            [
                pltpu.VMEM((bpw + LANES,), jnp.int32),
                pltpu.VMEM((bpw,), jnp.int32),
                pltpu.VMEM((bpw, T), jnp.int32),
                pltpu.VMEM((nflat + LANES,), jnp.int32),
                pltpu.VMEM((nrow, TILE_L), jnp.int32),
                pltpu.VMEM((bpw * D,), jnp.float32),
                pltpu.VMEM((bpw, D), jnp.float32),
                pltpu.VMEM((NBUF, T, 2 * D), jnp.float32),
                pltpu.VMEM((NBUF, T, D), jnp.float32),
                pltpu.SemaphoreType.DMA,
                pltpu.SemaphoreType.DMA,
            ]
            + [pltpu.SemaphoreType.DMA for _ in range(2 * NBUF)]
        ),
    )
    def emb(uid_h, gid_h, cid2_h, loff_h, cf_h, ct_h, gt_h, out_h,
            uid_v, gid_v, cid2_v, loff, eidx, s_v, g_v, in_b, out_b,
            sem_g, sem_s, *sems):
        gsem = sems[:NBUF]
        ssem = sems[NBUF:]
        wid = lax.axis_index("s") * NC + lax.axis_index("c")
        base = wid * bpw
        pltpu.sync_copy(uid_h.at[pl.ds(base, bpw)],
                        uid_v.at[pl.ds(0, bpw)])
        pltpu.sync_copy(gid_h.at[pl.ds(base, bpw)], gid_v)
        pltpu.sync_copy(cid2_h.at[pl.ds(base, bpw)], cid2_v)
        pltpu.sync_copy(loff_h.at[pl.ds(base * T, nflat)],
                        loff.at[pl.ds(0, nflat)])
        pltpu.async_copy(gt_h.at[gid_v], g_v, sem_g)

        iot = lax.iota(jnp.int32, LANES)

        def idxrow(r, carry):
            i0 = per_row * r
            for q in range(TILE_L // LANES):
                i_loc = i0 + q // (D // LANES)
                lane = lax.rem(uid_v[pl.ds(i_loc, LANES)][0], TILE_L)
                d_vec = (q % (D // LANES)) * LANES + iot
                eidx[r, pl.ds(q * LANES, LANES)] = (
                    (base + i_loc) * (D * TILE_L) + d_vec * TILE_L + lane
                )
            return carry

        lax.fori_loop(0, nrow, idxrow, 0)

        def efire(r, carry):
            pltpu.async_copy(cf_h.at[eidx.at[r]],
                             s_v.at[pl.ds(r * TILE_L, TILE_L)], sem_s)
            return carry

        def edrain(r, carry):
            pltpu.make_async_copy(cf_h.at[eidx.at[0]],
                                  s_v.at[pl.ds(0, TILE_L)], sem_s).wait()
            return carry

        lax.fori_loop(0, nrow, efire, 0)
        lax.fori_loop(0, nrow, edrain, 0)

        pltpu.make_async_copy(gt_h.at[gid_v], g_v, sem_g).wait()

        def srow(i, carry):
            for k in range(nk):
                s_v[pl.ds(i * D + k * LANES, LANES)] = (
                    s_v[pl.ds(i * D + k * LANES, LANES)]
                    + g_v[i, pl.ds(k * LANES, LANES)]
                )
            return carry

        lax.fori_loop(0, bpw, srow, 0)

        for j in range(NBUF):
            pltpu.async_copy(ct_h.at[cid2_v.at[j]], in_b.at[j], gsem[j])

        def group(gi, carry):
            for j in range(NBUF):
                b = gi * NBUF + j
                pltpu.make_async_copy(
                    ct_h.at[cid2_v.at[b]], in_b.at[j], gsem[j]).wait()

                @pl.when(gi > 0)
                def _():
                    pltpu.make_async_copy(
                        out_b.at[j], out_h.at[base + b], ssem[j]).wait()

                s_regs = [s_v[pl.ds(b * D + k * LANES, LANES)]
                          for k in range(nk)]

                def trow(t, c2):
                    hi = loff[pl.ds(b * T + t, LANES)][0] > 0
                    for k in range(nk):
                        v_lo = in_b[j, t, pl.ds(k * LANES, LANES)]
                        v_hi = in_b[j, t, pl.ds(D + k * LANES, LANES)]
                        out_b[j, t, pl.ds(k * LANES, LANES)] = (
                            jnp.where(hi, v_hi, v_lo) + s_regs[k]
                        )
                    return c2

                lax.fori_loop(0, T, trow, 0)

                pltpu.async_copy(out_b.at[j], out_h.at[base + b], ssem[j])

                @pl.when(b + NBUF < bpw)
                def _():
                    pltpu.async_copy(
                        ct_h.at[cid2_v.at[b + NBUF]], in_b.at[j], gsem[j])
            return carry

        lax.fori_loop(0, ngroups, group, 0)

        for j in range(NBUF):
            b = (ngroups - 1) * NBUF + j
            pltpu.make_async_copy(
                out_b.at[j], out_h.at[base + b], ssem[j]).wait()

    return emb


def kernel(user_id, creative_id, geo_id, T, user_table, creative_table,
           geo_table):
    del T
    B, t_static = creative_id.shape
    D = user_table.shape[1]
    chunks_k = _build_chunks(B, D, user_table.shape[0])
    uid = user_id.astype(jnp.int32)
    chunks = chunks_k(uid, user_table.T)
    repack, H = _build_repack(creative_table.shape[0], D)
    ct2 = repack(creative_table.T, creative_table.T)
    emb = _build_main(B, t_static, D, creative_table.shape[0],
                      geo_table.shape[0], H)
    cid = creative_id.astype(jnp.int32)
    big = (cid >= H).astype(jnp.int32)
    cid2 = cid - big * H
    loff = (big * D).reshape(-1)
    return emb(
        uid,
        geo_id.astype(jnp.int32),
        cid2,
        loff,
        chunks.reshape(-1),
        ct2,
        geo_table,
    )


# speedup vs baseline: 1.2392x; 1.2326x over previous
import functools

import jax
import jax.numpy as jnp
from jax import lax
from jax.experimental import pallas as pl
from jax.experimental.pallas import tpu as pltpu
from jax.experimental.pallas import tpu_sc as plsc

NBUF = 4
RING = 8
LEAD = 4
LANES = 16
TILE_L = 128


@functools.cache
def _build_chunks(B, D, VU):
    info = plsc.get_sparse_core_info()
    NC, NS = info.num_cores, info.num_subcores
    NW = NC * NS
    bpw = B // NW

    mesh = plsc.VectorSubcoreMesh(core_axis_name="c", subcore_axis_name="s")

    @functools.partial(
        pl.kernel,
        mesh=mesh,
        compiler_params=pltpu.CompilerParams(use_tc_tiling_on_sc=True),
        out_type=jax.ShapeDtypeStruct((B, D, TILE_L), jnp.float32),
        scratch_types=(
            [
                pltpu.VMEM((bpw + LANES,), jnp.int32),
                pltpu.VMEM((RING, D, TILE_L), jnp.float32),
            ]
            + [pltpu.SemaphoreType.DMA for _ in range(2 * RING)]
        ),
    )
    def chunks_k(uid_h, utT_h, c_h, uid_v, chunk, *sems):
        gsem = sems[:RING]
        wsem = sems[RING:]
        wid = lax.axis_index("s") * NC + lax.axis_index("c")
        base_i = wid * bpw
        pltpu.sync_copy(uid_h.at[pl.ds(base_i, bpw)],
                        uid_v.at[pl.ds(0, bpw)])

        def col(i):
            return (uid_v[pl.ds(i, LANES)][0] // TILE_L) * TILE_L

        for j in range(LEAD):
            pltpu.async_copy(utT_h.at[:, pl.ds(col(j), TILE_L)],
                             chunk.at[j], gsem[j])

        def step(i, carry):
            j_dyn = lax.rem(i, RING)
            for j in range(RING):

                @pl.when(j_dyn == j)
                def _():
                    pltpu.make_async_copy(
                        utT_h.at[:, pl.ds(0, TILE_L)], chunk.at[j],
                        gsem[j]).wait()
                    pltpu.async_copy(chunk.at[j], c_h.at[base_i + i],
                                     wsem[j])
                    jn = (j + LEAD) % RING

                    @pl.when(i >= LEAD)
                    def _():
                        pltpu.make_async_copy(
                            chunk.at[jn], c_h.at[base_i], wsem[jn]).wait()

                    @pl.when(i + LEAD < bpw)
                    def _():
                        pltpu.async_copy(
                            utT_h.at[:, pl.ds(col(i + LEAD), TILE_L)],
                            chunk.at[jn], gsem[jn])
            return carry

        lax.fori_loop(0, bpw, step, 0)
        for jj in range(LEAD):
            j = (bpw - LEAD + jj) % RING
            pltpu.make_async_copy(chunk.at[j], c_h.at[base_i],
                                  wsem[j]).wait()

    return chunks_k


@functools.cache
def _build_repack(VC, D):
    W = 1024
    G = -(-((VC + 1) // 2) // W)
    H = G * W

    last_blk = -(-VC // W) - 1

    def body(a_ref, b_ref, o_ref):
        o_ref[:, 0:D] = jnp.swapaxes(a_ref[...], 0, 1)
        o_ref[:, D:2 * D] = jnp.swapaxes(b_ref[...], 0, 1)

    rp = pl.pallas_call(
        body,
        grid=(G,),
        in_specs=[
            pl.BlockSpec((D, W), lambda k: (0, k)),
            pl.BlockSpec((D, W), lambda k: (0, jnp.minimum(k + G, last_blk))),
        ],
        out_specs=pl.BlockSpec((W, 2 * D), lambda k: (k, 0)),
        out_shape=jax.ShapeDtypeStruct((H, 2 * D), jnp.float32),
    )
    return rp, H


@functools.cache
def _build_main(B, T, D, VC, VG, H):
    info = plsc.get_sparse_core_info()
    NC, NS = info.num_cores, info.num_subcores
    NW = NC * NS
    assert B % NW == 0, (B, NW)
    bpw = B // NW
    assert bpw % NBUF == 0
    ngroups = bpw // NBUF
    nk = D // LANES
    nrow = bpw * D // TILE_L
    per_row = TILE_L // D
    nflat = bpw * T

    mesh = plsc.VectorSubcoreMesh(core_axis_name="c", subcore_axis_name="s")

    @functools.partial(
        pl.kernel,
        mesh=mesh,
        compiler_params=pltpu.CompilerParams(use_tc_tiling_on_sc=False),
        out_type=jax.ShapeDtypeStruct((B, T, D), jnp.float32),
        scratch_types=(
            [
                pltpu.VMEM((bpw + LANES,), jnp.int32),
                pltpu.VMEM((bpw,), jnp.int32),
                pltpu.VMEM((bpw, T), jnp.int32),
                pltpu.VMEM((nrow, TILE_L), jnp.int32),
                pltpu.VMEM((bpw * D,), jnp.float32),
                pltpu.VMEM((bpw, D), jnp.float32),
                pltpu.VMEM((NBUF, T, D), jnp.float32),
                pltpu.VMEM((NBUF, T, D), jnp.float32),
                pltpu.SemaphoreType.DMA,
                pltpu.SemaphoreType.DMA,
            ]
            + [pltpu.SemaphoreType.DMA for _ in range(2 * NBUF)]
        ),
    )
    def emb(uid_h, gid_h, cid2_h, cf_h, ct_h, gt_h, out_h,
            uid_v, gid_v, cid2_v, eidx, s_v, g_v, in_b, out_b,
            sem_g, sem_s, *sems):
        gsem = sems[:NBUF]
        ssem = sems[NBUF:]
        wid = lax.axis_index("s") * NC + lax.axis_index("c")
        base = wid * bpw
        pltpu.sync_copy(uid_h.at[pl.ds(base, bpw)],
                        uid_v.at[pl.ds(0, bpw)])
        pltpu.sync_copy(gid_h.at[pl.ds(base, bpw)], gid_v)
        pltpu.sync_copy(cid2_h.at[pl.ds(base, bpw)], cid2_v)
        pltpu.async_copy(gt_h.at[gid_v], g_v, sem_g)

        iot = lax.iota(jnp.int32, LANES)

        def idxrow(r, carry):
            i0 = per_row * r
            for q in range(TILE_L // LANES):
                i_loc = i0 + q // (D // LANES)
                lane = lax.rem(uid_v[pl.ds(i_loc, LANES)][0], TILE_L)
                d_vec = (q % (D // LANES)) * LANES + iot
                eidx[r, pl.ds(q * LANES, LANES)] = (
                    (base + i_loc) * (D * TILE_L) + d_vec * TILE_L + lane
                )
            return carry

        lax.fori_loop(0, nrow, idxrow, 0)

        def efire(r, carry):
            pltpu.async_copy(cf_h.at[eidx.at[r]],
                             s_v.at[pl.ds(r * TILE_L, TILE_L)], sem_s)
            return carry

        def edrain(r, carry):
            pltpu.make_async_copy(cf_h.at[eidx.at[0]],
                                  s_v.at[pl.ds(0, TILE_L)], sem_s).wait()
            return carry

        lax.fori_loop(0, nrow, efire, 0)
        lax.fori_loop(0, nrow, edrain, 0)

        pltpu.make_async_copy(gt_h.at[gid_v], g_v, sem_g).wait()

        def srow(i, carry):
            for k in range(nk):
                s_v[pl.ds(i * D + k * LANES, LANES)] = (
                    s_v[pl.ds(i * D + k * LANES, LANES)]
                    + g_v[i, pl.ds(k * LANES, LANES)]
                )
            return carry

        lax.fori_loop(0, bpw, srow, 0)

        for j in range(NBUF):
            pltpu.async_copy(ct_h.at[cid2_v.at[j]], in_b.at[j], gsem[j])

        def group(gi, carry):
            for j in range(NBUF):
                b = gi * NBUF + j
                pltpu.make_async_copy(
                    ct_h.at[cid2_v.at[b]], in_b.at[j], gsem[j]).wait()

                @pl.when(gi > 0)
                def _():
                    pltpu.make_async_copy(
                        out_b.at[j], out_h.at[base + b], ssem[j]).wait()

                s_regs = [s_v[pl.ds(b * D + k * LANES, LANES)]
                          for k in range(nk)]

                def trow(t, c2):
                    for k in range(nk):
                        out_b[j, t, pl.ds(k * LANES, LANES)] = (
                            in_b[j, t, pl.ds(k * LANES, LANES)] + s_regs[k]
                        )
                    return c2

                lax.fori_loop(0, T, trow, 0)

                pltpu.async_copy(out_b.at[j], out_h.at[base + b], ssem[j])

                @pl.when(b + NBUF < bpw)
                def _():
                    pltpu.async_copy(
                        ct_h.at[cid2_v.at[b + NBUF]], in_b.at[j], gsem[j])
            return carry

        lax.fori_loop(0, ngroups, group, 0)

        for j in range(NBUF):
            b = (ngroups - 1) * NBUF + j
            pltpu.make_async_copy(
                out_b.at[j], out_h.at[base + b], ssem[j]).wait()

    return emb


def kernel(user_id, creative_id, geo_id, T, user_table, creative_table,
           geo_table):
    del T
    B, t_static = creative_id.shape
    D = user_table.shape[1]
    chunks_k = _build_chunks(B, D, user_table.shape[0])
    uid = user_id.astype(jnp.int32)
    chunks = chunks_k(uid, user_table.T)
    repack, H = _build_repack(creative_table.shape[0], D)
    ct2 = repack(creative_table.T, creative_table.T)
    emb = _build_main(B, t_static, D, creative_table.shape[0],
                      geo_table.shape[0], H)
    cid = creative_id.astype(jnp.int32)
    big = (cid >= H).astype(jnp.int32)
    cid2 = 2 * (cid - big * H) + big
    return emb(
        uid,
        geo_id.astype(jnp.int32),
        cid2,
        chunks.reshape(-1),
        ct2.reshape(2 * H, D),
        geo_table,
    )


# speedup vs baseline: 1.2450x; 1.0047x over previous
import functools

import jax
import jax.numpy as jnp
from jax import lax
from jax.experimental import pallas as pl
from jax.experimental.pallas import tpu as pltpu
from jax.experimental.pallas import tpu_sc as plsc

NBUF = 4
RING = 8
LEAD = 4
LANES = 16
TILE_L = 128


@functools.cache
def _build_chunks(B, D, VU):
    info = plsc.get_sparse_core_info()
    NC, NS = info.num_cores, info.num_subcores
    NW = NC * NS
    bpw = B // NW

    mesh = plsc.VectorSubcoreMesh(core_axis_name="c", subcore_axis_name="s")

    @functools.partial(
        pl.kernel,
        mesh=mesh,
        compiler_params=pltpu.CompilerParams(use_tc_tiling_on_sc=True),
        out_type=jax.ShapeDtypeStruct((B, D, TILE_L), jnp.float32),
        scratch_types=(
            [
                pltpu.VMEM((bpw + LANES,), jnp.int32),
                pltpu.VMEM((RING, D, TILE_L), jnp.float32),
            ]
            + [pltpu.SemaphoreType.DMA for _ in range(2 * RING)]
        ),
    )
    def chunks_k(uid_h, utT_h, c_h, uid_v, chunk, *sems):
        gsem = sems[:RING]
        wsem = sems[RING:]
        wid = lax.axis_index("s") * NC + lax.axis_index("c")
        base_i = wid * bpw
        pltpu.sync_copy(uid_h.at[pl.ds(base_i, bpw)],
                        uid_v.at[pl.ds(0, bpw)])

        def col(i):
            return (uid_v[pl.ds(i, LANES)][0] // TILE_L) * TILE_L

        for j in range(LEAD):
            pltpu.async_copy(utT_h.at[:, pl.ds(col(j), TILE_L)],
                             chunk.at[j], gsem[j])

        def step(i, carry):
            j_dyn = lax.rem(i, RING)
            for j in range(RING):

                @pl.when(j_dyn == j)
                def _():
                    pltpu.make_async_copy(
                        utT_h.at[:, pl.ds(0, TILE_L)], chunk.at[j],
                        gsem[j]).wait()
                    pltpu.async_copy(chunk.at[j], c_h.at[base_i + i],
                                     wsem[j])
                    jn = (j + LEAD) % RING

                    @pl.when(i >= LEAD)
                    def _():
                        pltpu.make_async_copy(
                            chunk.at[jn], c_h.at[base_i], wsem[jn]).wait()

                    @pl.when(i + LEAD < bpw)
                    def _():
                        pltpu.async_copy(
                            utT_h.at[:, pl.ds(col(i + LEAD), TILE_L)],
                            chunk.at[jn], gsem[jn])
            return carry

        lax.fori_loop(0, bpw, step, 0)
        for jj in range(LEAD):
            j = (bpw - LEAD + jj) % RING
            pltpu.make_async_copy(chunk.at[j], c_h.at[base_i],
                                  wsem[j]).wait()

    return chunks_k


@functools.cache
def _build_repack(VC, D):
    W = 1024
    G = -(-((VC + 1) // 2) // W)
    H = G * W

    last_blk = -(-VC // W) - 1

    def body(a_ref, b_ref, o_ref):
        o_ref[:, 0:D] = jnp.swapaxes(a_ref[...], 0, 1)
        o_ref[:, D:2 * D] = jnp.swapaxes(b_ref[...], 0, 1)

    rp = pl.pallas_call(
        body,
        grid=(G,),
        in_specs=[
            pl.BlockSpec((D, W), lambda k: (0, k)),
            pl.BlockSpec((D, W), lambda k: (0, jnp.minimum(k + G, last_blk))),
        ],
        out_specs=pl.BlockSpec((W, 2 * D), lambda k: (k, 0)),
        out_shape=jax.ShapeDtypeStruct((H, 2 * D), jnp.float32),
    )
    return rp, H


@functools.cache
def _build_main(B, T, D, VC, VG, H):
    info = plsc.get_sparse_core_info()
    NC, NS = info.num_cores, info.num_subcores
    NW = NC * NS
    assert B % NW == 0, (B, NW)
    bpw = B // NW
    assert bpw % NBUF == 0
    ngroups = bpw // NBUF
    nk = D // LANES
    nrow = bpw * D // TILE_L
    per_row = TILE_L // D
    nflat = bpw * T

    mesh = plsc.VectorSubcoreMesh(core_axis_name="c", subcore_axis_name="s")

    @functools.partial(
        pl.kernel,
        mesh=mesh,
        compiler_params=pltpu.CompilerParams(use_tc_tiling_on_sc=False),
        out_type=jax.ShapeDtypeStruct((B, T, D), jnp.float32),
        scratch_types=(
            [
                pltpu.VMEM((bpw + LANES,), jnp.int32),
                pltpu.VMEM((bpw,), jnp.int32),
                pltpu.VMEM((bpw, T), jnp.int32),
                pltpu.VMEM((nrow, TILE_L), jnp.int32),
                pltpu.VMEM((bpw * D,), jnp.float32),
                pltpu.VMEM((bpw, D), jnp.float32),
                pltpu.VMEM((NBUF, T, D), jnp.float32),
                pltpu.VMEM((NBUF, T, D), jnp.float32),
                pltpu.SemaphoreType.DMA,
                pltpu.SemaphoreType.DMA,
            ]
            + [pltpu.SemaphoreType.DMA for _ in range(2 * NBUF)]
        ),
    )
    def emb(uid_h, gid_h, cid2_h, cf_h, ct_h, gt_h, out_h,
            uid_v, gid_v, cid2_v, eidx, s_v, g_v, in_b, out_b,
            sem_g, sem_s, *sems):
        gsem = sems[:NBUF]
        ssem = sems[NBUF:]
        wid = lax.axis_index("s") * NC + lax.axis_index("c")
        base = wid * bpw
        pltpu.sync_copy(uid_h.at[pl.ds(base, bpw)],
                        uid_v.at[pl.ds(0, bpw)])
        pltpu.sync_copy(gid_h.at[pl.ds(base, bpw)], gid_v)
        pltpu.sync_copy(cid2_h.at[pl.ds(base, bpw)], cid2_v)
        pltpu.async_copy(gt_h.at[gid_v], g_v, sem_g)

        iot = lax.iota(jnp.int32, LANES)

        def idxrow(r, carry):
            i0 = per_row * r
            for q in range(TILE_L // LANES):
                i_loc = i0 + q // (D // LANES)
                lane = lax.rem(uid_v[pl.ds(i_loc, LANES)][0], TILE_L)
                d_vec = (q % (D // LANES)) * LANES + iot
                eidx[r, pl.ds(q * LANES, LANES)] = (
                    (base + i_loc) * (D * TILE_L) + d_vec * TILE_L + lane
                )
            return carry

        lax.fori_loop(0, nrow, idxrow, 0)

        def efire(r, carry):
            pltpu.async_copy(cf_h.at[eidx.at[r]],
                             s_v.at[pl.ds(r * TILE_L, TILE_L)], sem_s)
            return carry

        def edrain(r, carry):
            pltpu.make_async_copy(cf_h.at[eidx.at[0]],
                                  s_v.at[pl.ds(0, TILE_L)], sem_s).wait()
            return carry

        lax.fori_loop(0, nrow, efire, 0)
        lax.fori_loop(0, nrow, edrain, 0)

        pltpu.make_async_copy(gt_h.at[gid_v], g_v, sem_g).wait()

        def srow(i, carry):
            for k in range(nk):
                s_v[pl.ds(i * D + k * LANES, LANES)] = (
                    s_v[pl.ds(i * D + k * LANES, LANES)]
                    + g_v[i, pl.ds(k * LANES, LANES)]
                )
            return carry

        lax.fori_loop(0, bpw, srow, 0)

        for j in range(NBUF):
            pltpu.async_copy(ct_h.at[cid2_v.at[j]], in_b.at[j], gsem[j])

        def group(gi, carry):
            for j in range(NBUF):
                b = gi * NBUF + j
                pltpu.make_async_copy(
                    ct_h.at[cid2_v.at[b]], in_b.at[j], gsem[j]).wait()

                @pl.when(gi > 0)
                def _():
                    pltpu.make_async_copy(
                        out_b.at[j], out_h.at[base + b], ssem[j]).wait()

                s_regs = [s_v[pl.ds(b * D + k * LANES, LANES)]
                          for k in range(nk)]

                def trow(t, c2):
                    for k in range(nk):
                        out_b[j, t, pl.ds(k * LANES, LANES)] = (
                            in_b[j, t, pl.ds(k * LANES, LANES)] + s_regs[k]
                        )
                    return c2

                lax.fori_loop(0, T, trow, 0)

                pltpu.async_copy(out_b.at[j], out_h.at[base + b], ssem[j])

                @pl.when(b + NBUF < bpw)
                def _():
                    pltpu.async_copy(
                        ct_h.at[cid2_v.at[b + NBUF]], in_b.at[j], gsem[j])
            return carry

        lax.fori_loop(0, ngroups, group, 0)

        for j in range(NBUF):
            b = (ngroups - 1) * NBUF + j
            pltpu.make_async_copy(
                out_b.at[j], out_h.at[base + b], ssem[j]).wait()

    return emb


def kernel(user_id, creative_id, geo_id, T, user_table, creative_table,
           geo_table):
    del T
    B, t_static = creative_id.shape
    D = user_table.shape[1]
    chunks_k = _build_chunks(B, D, user_table.shape[0])
    uid = user_id.astype(jnp.int32)
    chunks = chunks_k(uid, user_table.T)
    repack, H = _build_repack(creative_table.shape[0], D)
    ct2 = repack(creative_table.T, creative_table.T)
    repack_g, Hg = _build_repack(geo_table.shape[0], D)
    gt2 = repack_g(geo_table.T, geo_table.T)
    emb = _build_main(B, t_static, D, creative_table.shape[0],
                      geo_table.shape[0], H)
    cid = creative_id.astype(jnp.int32)
    big = (cid >= H).astype(jnp.int32)
    cid2 = 2 * (cid - big * H) + big
    gid = geo_id.astype(jnp.int32)
    gbig = (gid >= Hg).astype(jnp.int32)
    gid2 = 2 * (gid - gbig * Hg) + gbig
    return emb(
        uid,
        gid2,
        cid2,
        chunks.reshape(-1),
        ct2.reshape(2 * H, D),
        gt2.reshape(2 * Hg, D),
    )


# speedup vs baseline: 1.2576x; 1.0101x over previous
import functools

import jax
import jax.numpy as jnp
from jax import lax
from jax.experimental import pallas as pl
from jax.experimental.pallas import tpu as pltpu
from jax.experimental.pallas import tpu_sc as plsc

NBUF = 8
RING = 8
LEAD = 4
LANES = 16
TILE_L = 128


@functools.cache
def _build_chunks(B, D, VU):
    info = plsc.get_sparse_core_info()
    NC, NS = info.num_cores, info.num_subcores
    NW = NC * NS
    bpw = B // NW

    mesh = plsc.VectorSubcoreMesh(core_axis_name="c", subcore_axis_name="s")

    @functools.partial(
        pl.kernel,
        mesh=mesh,
        compiler_params=pltpu.CompilerParams(use_tc_tiling_on_sc=True),
        out_type=jax.ShapeDtypeStruct((B, D, TILE_L), jnp.float32),
        scratch_types=(
            [
                pltpu.VMEM((bpw + LANES,), jnp.int32),
                pltpu.VMEM((RING, D, TILE_L), jnp.float32),
            ]
            + [pltpu.SemaphoreType.DMA for _ in range(2 * RING)]
        ),
    )
    def chunks_k(uid_h, utT_h, c_h, uid_v, chunk, *sems):
        gsem = sems[:RING]
        wsem = sems[RING:]
        wid = lax.axis_index("s") * NC + lax.axis_index("c")
        base_i = wid * bpw
        pltpu.sync_copy(uid_h.at[pl.ds(base_i, bpw)],
                        uid_v.at[pl.ds(0, bpw)])

        def col(i):
            return (uid_v[pl.ds(i, LANES)][0] // TILE_L) * TILE_L

        for j in range(LEAD):
            pltpu.async_copy(utT_h.at[:, pl.ds(col(j), TILE_L)],
                             chunk.at[j], gsem[j])

        def step(i, carry):
            j_dyn = lax.rem(i, RING)
            for j in range(RING):

                @pl.when(j_dyn == j)
                def _():
                    pltpu.make_async_copy(
                        utT_h.at[:, pl.ds(0, TILE_L)], chunk.at[j],
                        gsem[j]).wait()
                    pltpu.async_copy(chunk.at[j], c_h.at[base_i + i],
                                     wsem[j])
                    jn = (j + LEAD) % RING

                    @pl.when(i >= LEAD)
                    def _():
                        pltpu.make_async_copy(
                            chunk.at[jn], c_h.at[base_i], wsem[jn]).wait()

                    @pl.when(i + LEAD < bpw)
                    def _():
                        pltpu.async_copy(
                            utT_h.at[:, pl.ds(col(i + LEAD), TILE_L)],
                            chunk.at[jn], gsem[jn])
            return carry

        lax.fori_loop(0, bpw, step, 0)
        for jj in range(LEAD):
            j = (bpw - LEAD + jj) % RING
            pltpu.make_async_copy(chunk.at[j], c_h.at[base_i],
                                  wsem[j]).wait()

    return chunks_k


@functools.cache
def _build_repack(VC, D):
    W = 1024
    G = -(-((VC + 1) // 2) // W)
    H = G * W

    last_blk = -(-VC // W) - 1

    def body(a_ref, b_ref, o_ref):
        o_ref[:, 0:D] = jnp.swapaxes(a_ref[...], 0, 1)
        o_ref[:, D:2 * D] = jnp.swapaxes(b_ref[...], 0, 1)

    rp = pl.pallas_call(
        body,
        grid=(G,),
        in_specs=[
            pl.BlockSpec((D, W), lambda k: (0, k)),
            pl.BlockSpec((D, W), lambda k: (0, jnp.minimum(k + G, last_blk))),
        ],
        out_specs=pl.BlockSpec((W, 2 * D), lambda k: (k, 0)),
        out_shape=jax.ShapeDtypeStruct((H, 2 * D), jnp.float32),
    )
    return rp, H


@functools.cache
def _build_main(B, T, D, VC, VG, H):
    info = plsc.get_sparse_core_info()
    NC, NS = info.num_cores, info.num_subcores
    NW = NC * NS
    assert B % NW == 0, (B, NW)
    bpw = B // NW
    assert bpw % NBUF == 0
    ngroups = bpw // NBUF
    nk = D // LANES
    nrow = bpw * D // TILE_L
    per_row = TILE_L // D
    nflat = bpw * T

    mesh = plsc.VectorSubcoreMesh(core_axis_name="c", subcore_axis_name="s")

    @functools.partial(
        pl.kernel,
        mesh=mesh,
        compiler_params=pltpu.CompilerParams(use_tc_tiling_on_sc=False),
        out_type=jax.ShapeDtypeStruct((B, T, D), jnp.float32),
        scratch_types=(
            [
                pltpu.VMEM((bpw + LANES,), jnp.int32),
                pltpu.VMEM((bpw,), jnp.int32),
                pltpu.VMEM((bpw, T), jnp.int32),
                pltpu.VMEM((nrow, TILE_L), jnp.int32),
                pltpu.VMEM((bpw * D,), jnp.float32),
                pltpu.VMEM((bpw, D), jnp.float32),
                pltpu.VMEM((NBUF, T, D), jnp.float32),
                pltpu.VMEM((NBUF, T, D), jnp.float32),
                pltpu.SemaphoreType.DMA,
                pltpu.SemaphoreType.DMA,
            ]
            + [pltpu.SemaphoreType.DMA for _ in range(2 * NBUF)]
        ),
    )
    def emb(uid_h, gid_h, cid2_h, cf_h, ct_h, gt_h, out_h,
            uid_v, gid_v, cid2_v, eidx, s_v, g_v, in_b, out_b,
            sem_g, sem_s, *sems):
        gsem = sems[:NBUF]
        ssem = sems[NBUF:]
        wid = lax.axis_index("s") * NC + lax.axis_index("c")
        base = wid * bpw
        pltpu.sync_copy(uid_h.at[pl.ds(base, bpw)],
                        uid_v.at[pl.ds(0, bpw)])
        pltpu.sync_copy(gid_h.at[pl.ds(base, bpw)], gid_v)
        pltpu.sync_copy(cid2_h.at[pl.ds(base, bpw)], cid2_v)
        for j in range(NBUF):
            pltpu.async_copy(ct_h.at[cid2_v.at[j]], in_b.at[j], gsem[j])
        pltpu.async_copy(gt_h.at[gid_v], g_v, sem_g)

        iot = lax.iota(jnp.int32, LANES)

        def idxrow(r, carry):
            i0 = per_row * r
            for q in range(TILE_L // LANES):
                i_loc = i0 + q // (D // LANES)
                lane = lax.rem(uid_v[pl.ds(i_loc, LANES)][0], TILE_L)
                d_vec = (q % (D // LANES)) * LANES + iot
                eidx[r, pl.ds(q * LANES, LANES)] = (
                    (base + i_loc) * (D * TILE_L) + d_vec * TILE_L + lane
                )
            return carry

        lax.fori_loop(0, nrow, idxrow, 0)

        def efire(r, carry):
            pltpu.async_copy(cf_h.at[eidx.at[r]],
                             s_v.at[pl.ds(r * TILE_L, TILE_L)], sem_s)
            return carry

        def edrain(r, carry):
            pltpu.make_async_copy(cf_h.at[eidx.at[0]],
                                  s_v.at[pl.ds(0, TILE_L)], sem_s).wait()
            return carry

        lax.fori_loop(0, nrow, efire, 0)
        lax.fori_loop(0, nrow, edrain, 0)

        pltpu.make_async_copy(gt_h.at[gid_v], g_v, sem_g).wait()

        def srow(i, carry):
            for k in range(nk):
                s_v[pl.ds(i * D + k * LANES, LANES)] = (
                    s_v[pl.ds(i * D + k * LANES, LANES)]
                    + g_v[i, pl.ds(k * LANES, LANES)]
                )
            return carry

        lax.fori_loop(0, bpw, srow, 0)

        def group(gi, carry):
            for j in range(NBUF):
                b = gi * NBUF + j
                pltpu.make_async_copy(
                    ct_h.at[cid2_v.at[b]], in_b.at[j], gsem[j]).wait()

                @pl.when(gi > 0)
                def _():
                    pltpu.make_async_copy(
                        out_b.at[j], out_h.at[base + b], ssem[j]).wait()

                s_regs = [s_v[pl.ds(b * D + k * LANES, LANES)]
                          for k in range(nk)]

                def trow(t, c2):
                    for k in range(nk):
                        out_b[j, t, pl.ds(k * LANES, LANES)] = (
                            in_b[j, t, pl.ds(k * LANES, LANES)] + s_regs[k]
                        )
                    return c2

                lax.fori_loop(0, T, trow, 0)

                pltpu.async_copy(out_b.at[j], out_h.at[base + b], ssem[j])

                @pl.when(b + NBUF < bpw)
                def _():
                    pltpu.async_copy(
                        ct_h.at[cid2_v.at[b + NBUF]], in_b.at[j], gsem[j])
            return carry

        lax.fori_loop(0, ngroups, group, 0)

        for j in range(NBUF):
            b = (ngroups - 1) * NBUF + j
            pltpu.make_async_copy(
                out_b.at[j], out_h.at[base + b], ssem[j]).wait()

    return emb


def kernel(user_id, creative_id, geo_id, T, user_table, creative_table,
           geo_table):
    del T
    B, t_static = creative_id.shape
    D = user_table.shape[1]
    chunks_k = _build_chunks(B, D, user_table.shape[0])
    uid = user_id.astype(jnp.int32)
    chunks = chunks_k(uid, user_table.T)
    repack, H = _build_repack(creative_table.shape[0], D)
    ct2 = repack(creative_table.T, creative_table.T)
    repack_g, Hg = _build_repack(geo_table.shape[0], D)
    gt2 = repack_g(geo_table.T, geo_table.T)
    emb = _build_main(B, t_static, D, creative_table.shape[0],
                      geo_table.shape[0], H)
    cid = creative_id.astype(jnp.int32)
    big = (cid >= H).astype(jnp.int32)
    cid2 = 2 * (cid - big * H) + big
    gid = geo_id.astype(jnp.int32)
    gbig = (gid >= Hg).astype(jnp.int32)
    gid2 = 2 * (gid - gbig * Hg) + gbig
    return emb(
        uid,
        gid2,
        cid2,
        chunks.reshape(-1),
        ct2.reshape(2 * H, D),
        gt2.reshape(2 * Hg, D),
    )
